# baseline (speedup 1.0000x reference)
import math

import jax
import jax.numpy as jnp
from jax.experimental import pallas as pl

_M, _K, _D = 8, 256, 32
_HW = 256
_EPS = 1e-05
_C = 0.5 * math.log(2.0 * math.pi)
_DELTA = 1.2e-4
_S = 128
_NEG = -1e30


def _stage1_body(x_ref, mus_ref, musbf_ref, sc_ref, lp_ref,
                 out_ref, cand_ref, gap_ref, kld_ref):
    b = pl.program_id(0)
    m = pl.program_id(1)

    xs = x_ref[0]
    mus = mus_ref[0]
    musbf = musbf_ref[0]
    sc = jnp.clip(sc_ref[0], _EPS, None)
    lp = lp_ref[0]

    a = 0.5 / (sc * sc)
    musq = jnp.sum(mus * mus, axis=1, keepdims=True)
    cst = -_D * (jnp.log(sc) + _C) + lp

    dot = jnp.dot(mus, xs, preferred_element_type=jnp.float32,
                  precision=jax.lax.Precision.HIGHEST)
    logits = a * (2.0 * dot - musq) + cst

    colmax = jnp.max(logits, axis=0, keepdims=True)
    shifted = logits - colmax
    e = jnp.exp(shifted)
    se = jnp.sum(e, axis=0, keepdims=True)
    lse_sh = jnp.log(se)
    p = e / se
    lpmax = jnp.max(lp)
    log_prior = lp - (lpmax + jnp.log(jnp.sum(jnp.exp(lp - lpmax))))
    kld_part = jnp.sum(p * (shifted - lse_sh - log_prior))

    iota = jax.lax.broadcasted_iota(jnp.int32, (_K, _HW), 0)
    lcur = logits
    idxs = []
    vals = []
    for _ in range(4):
        vmax = jnp.max(lcur, axis=0, keepdims=True)
        idx = jnp.min(jnp.where(lcur == vmax, iota, _K), axis=0, keepdims=True)
        idxs.append(idx)
        vals.append(vmax)
        lcur = jnp.where(iota == idx, _NEG, lcur)

    cand_ref[0] = jnp.concatenate(idxs, axis=0)
    gap_ref[0] = vals[0] - vals[1]

    onehot = (iota == idxs[0]).astype(jnp.float32)
    sampleT = jax.lax.dot_general(
        musbf, onehot, (((0,), (0,)), ((), ())),
        preferred_element_type=jnp.float32,
        precision=jax.lax.Precision.HIGHEST)
    out_ref[0] = sampleT

    @pl.when(jnp.logical_and(b == 0, m == 0))
    def _init():
        kld_ref[...] = jnp.zeros((1, 1), jnp.float32)
    kld_ref[...] += jnp.reshape(kld_part, (1, 1))


def _stage2_body(xg_ref, musg_ref, musbfg_ref, den_ref, lsc_ref, lpg_ref,
                 kc_ref, row_ref):
    xg = xg_ref[...]
    musg = musg_ref[...]
    diff = xg - musg
    w = -(diff * diff) / den_ref[...] - lsc_ref[...] - _C

    parts = []
    for c in range(4):
        t = w[:, 8 * c:8 * c + 8]
        t = t[:, 0:4] + t[:, 4:8]
        t = t[:, 0:2] + t[:, 2:4]
        t = t[:, 0:1] + t[:, 1:2]
        parts.append(t)
    s = ((parts[0] + parts[1]) + parts[2]) + parts[3]
    v = s + lpg_ref[...]

    kc = kc_ref[...]
    musbfg = musbfg_ref[...]
    bv = v[0:_S]
    bk = kc[0:_S]
    brow = musbfg[0:_S]
    for j in range(1, 4):
        vj = v[j * _S:(j + 1) * _S]
        kj = kc[j * _S:(j + 1) * _S]
        rj = musbfg[j * _S:(j + 1) * _S]
        better = jnp.logical_or(vj > bv,
                                jnp.logical_and(vj == bv, kj < bk))
        bv = jnp.where(better, vj, bv)
        bk = jnp.where(better, kj, bk)
        brow = jnp.where(jnp.broadcast_to(better, (_S, _D)), rj, brow)
    row_ref[...] = brow


@jax.jit
def kernel(x, mus, scales, log_py_raw):
    b, c, h, w = x.shape
    hw = h * w
    x3 = x.reshape(b, c, hw)
    sc3 = scales.reshape(_M, _K, 1)
    lp3 = log_py_raw.reshape(_M, _K, 1)
    musbf = mus.astype(jnp.bfloat16).astype(jnp.float32)

    sample3, cand, gap, kld_acc = pl.pallas_call(
        _stage1_body,
        grid=(b, _M),
        in_specs=[
            pl.BlockSpec((1, _D, hw), lambda bi, mi: (bi, mi, 0)),
            pl.BlockSpec((1, _K, _D), lambda bi, mi: (mi, 0, 0)),
            pl.BlockSpec((1, _K, _D), lambda bi, mi: (mi, 0, 0)),
            pl.BlockSpec((1, _K, 1), lambda bi, mi: (mi, 0, 0)),
            pl.BlockSpec((1, _K, 1), lambda bi, mi: (mi, 0, 0)),
        ],
        out_specs=[
            pl.BlockSpec((1, _D, hw), lambda bi, mi: (bi, mi, 0)),
            pl.BlockSpec((1, 4, hw), lambda bi, mi: (bi * _M + mi, 0, 0)),
            pl.BlockSpec((1, 1, hw), lambda bi, mi: (bi * _M + mi, 0, 0)),
            pl.BlockSpec((1, 1), lambda bi, mi: (0, 0)),
        ],
        out_shape=[
            jax.ShapeDtypeStruct((b, c, hw), jnp.float32),
            jax.ShapeDtypeStruct((b * _M, 4, hw), jnp.int32),
            jax.ShapeDtypeStruct((b * _M, 1, hw), jnp.float32),
            jax.ShapeDtypeStruct((1, 1), jnp.float32),
        ],
    )(x3, mus, musbf, sc3, lp3)

    riskflat = (gap[:, 0, :] < _DELTA).reshape(-1)
    rows = jnp.nonzero(riskflat, size=_S, fill_value=0)[0]
    bm_s = rows // hw
    t_s = rows % hw
    b_s = bm_s // _M
    m_s = bm_s % _M

    cands = cand[bm_s, :, t_s]
    cm = cands.T.reshape(-1)
    m_s4 = jnp.tile(m_s, 4)
    t_s4 = jnp.tile(t_s, 4)
    b_s4 = jnp.tile(b_s, 4)

    dcols = m_s4[:, None] * _D + jnp.arange(_D)[None, :]
    xg4 = x3[b_s4[:, None], dcols, t_s4[:, None]]
    musg = mus[m_s4, cm, :]
    musbfg = musbf[m_s4, cm, :]
    scck = jnp.clip(scales, _EPS, None)
    den = (2.0 * scck ** 2)[m_s4, cm, :]
    lsc = jnp.log(scck)[m_s4, cm, :]
    lpg = log_py_raw[m_s4, cm][:, None]

    win_rows = pl.pallas_call(
        _stage2_body,
        out_shape=jax.ShapeDtypeStruct((_S, _D), jnp.float32),
    )(xg4, musg, musbfg, den, lsc, lpg, cm.reshape(-1, 1))

    dcols_s = m_s[:, None] * _D + jnp.arange(_D)[None, :]
    sample3 = sample3.at[b_s[:, None], dcols_s, t_s[:, None]].set(win_rows)

    sample = sample3.reshape(b, c, h, w)
    kldesum = kld_acc[0, 0] / b
    return (sample, kldesum, jnp.zeros_like(kldesum))


# speedup vs baseline: 1.7708x; 1.7708x over previous
import math

import jax
import jax.numpy as jnp
from jax.experimental import pallas as pl

_M, _K, _D = 8, 256, 32
_HW = 256
_EPS = 1e-05
_C = 0.5 * math.log(2.0 * math.pi)
_DELTA = 1.2e-4
_S = 128
_NEG = -1e30


def _stage1_body(x_ref, mus_ref, musbf_ref, sc_ref, lp_ref,
                 out_ref, cand_ref, gap_ref, kld_ref):
    b = pl.program_id(0)
    m = pl.program_id(1)

    xs = x_ref[0]
    mus = mus_ref[0]
    musbf = musbf_ref[0]
    sc = jnp.clip(sc_ref[0], _EPS, None)
    lp = lp_ref[0]

    a = 0.5 / (sc * sc)
    musq = jnp.sum(mus * mus, axis=1, keepdims=True)
    cst = -_D * (jnp.log(sc) + _C) + lp

    dot = jnp.dot(mus, xs, preferred_element_type=jnp.float32,
                  precision=jax.lax.Precision.HIGHEST)
    logits = a * (2.0 * dot - musq) + cst

    colmax = jnp.max(logits, axis=0, keepdims=True)
    shifted = logits - colmax
    e = jnp.exp(shifted)
    se = jnp.sum(e, axis=0, keepdims=True)
    lse_sh = jnp.log(se)
    p = e / se
    lpmax = jnp.max(lp)
    log_prior = lp - (lpmax + jnp.log(jnp.sum(jnp.exp(lp - lpmax))))
    kld_part = jnp.sum(p * (shifted - lse_sh - log_prior))

    iota = jax.lax.broadcasted_iota(jnp.int32, (_K, _HW), 0)
    lcur = logits
    idxs = []
    vals = []
    for _ in range(4):
        vmax = jnp.max(lcur, axis=0, keepdims=True)
        idx = jnp.min(jnp.where(lcur == vmax, iota, _K), axis=0, keepdims=True)
        idxs.append(idx)
        vals.append(vmax)
        lcur = jnp.where(iota == idx, _NEG, lcur)

    cand_ref[0] = jnp.concatenate(idxs, axis=0)
    gap_ref[0] = vals[0] - vals[1]

    onehot = (iota == idxs[0]).astype(jnp.float32)
    sampleT = jax.lax.dot_general(
        musbf, onehot, (((0,), (0,)), ((), ())),
        preferred_element_type=jnp.float32,
        precision=jax.lax.Precision.HIGHEST)
    out_ref[0] = sampleT

    @pl.when(jnp.logical_and(b == 0, m == 0))
    def _init():
        kld_ref[...] = jnp.zeros((1, 1), jnp.float32)
    kld_ref[...] += jnp.reshape(kld_part, (1, 1))


def _ref_tree_sum(w):
    parts = []
    for c in range(4):
        t = w[:, 8 * c:8 * c + 8]
        t = t[:, 0:4] + t[:, 4:8]
        t = t[:, 0:2] + t[:, 2:4]
        t = t[:, 0:1] + t[:, 1:2]
        parts.append(t)
    return ((parts[0] + parts[1]) + parts[2]) + parts[3]


def _stage2_body(xrow_ref, msel_ref, musg_ref, musbfg_ref, aux_ref,
                 kc_ref, row_ref):
    xrow = xrow_ref[...]
    msel = msel_ref[...]
    xg = jnp.zeros((_S, _D), jnp.float32)
    for m in range(_M):
        pick = (msel == m).astype(jnp.float32)
        xg = xg + xrow[:, m * _D:(m + 1) * _D] * pick

    bv = bk = brow = None
    for j in range(4):
        sl = slice(j * _S, (j + 1) * _S)
        musg = musg_ref[sl, :]
        aux = aux_ref[sl, :]
        diff = xg - musg
        w = -(diff * diff) / aux[:, 0:1] - aux[:, 1:2] - _C
        v = _ref_tree_sum(w) + aux[:, 2:3]
        kj = kc_ref[sl, :]
        rj = musbfg_ref[sl, :]
        if j == 0:
            bv, bk, brow = v, kj, rj
        else:
            better = jnp.logical_or(v > bv,
                                    jnp.logical_and(v == bv, kj < bk))
            bv = jnp.where(better, v, bv)
            bk = jnp.where(better, kj, bk)
            brow = jnp.where(jnp.broadcast_to(better, (_S, _D)), rj, brow)
    row_ref[...] = brow


@jax.jit
def kernel(x, mus, scales, log_py_raw):
    b, c, h, w = x.shape
    hw = h * w
    x3 = x.reshape(b, c, hw)
    sc3 = scales.reshape(_M, _K, 1)
    lp3 = log_py_raw.reshape(_M, _K, 1)
    musbf = mus.astype(jnp.bfloat16).astype(jnp.float32)

    sample3, cand, gap, kld_acc = pl.pallas_call(
        _stage1_body,
        grid=(b, _M),
        in_specs=[
            pl.BlockSpec((1, _D, hw), lambda bi, mi: (bi, mi, 0)),
            pl.BlockSpec((1, _K, _D), lambda bi, mi: (mi, 0, 0)),
            pl.BlockSpec((1, _K, _D), lambda bi, mi: (mi, 0, 0)),
            pl.BlockSpec((1, _K, 1), lambda bi, mi: (mi, 0, 0)),
            pl.BlockSpec((1, _K, 1), lambda bi, mi: (mi, 0, 0)),
        ],
        out_specs=[
            pl.BlockSpec((1, _D, hw), lambda bi, mi: (bi, mi, 0)),
            pl.BlockSpec((1, 4, hw), lambda bi, mi: (bi * _M + mi, 0, 0)),
            pl.BlockSpec((1, 1, hw), lambda bi, mi: (bi * _M + mi, 0, 0)),
            pl.BlockSpec((1, 1), lambda bi, mi: (0, 0)),
        ],
        out_shape=[
            jax.ShapeDtypeStruct((b, c, hw), jnp.float32),
            jax.ShapeDtypeStruct((b * _M, 4, hw), jnp.int32),
            jax.ShapeDtypeStruct((b * _M, 1, hw), jnp.float32),
            jax.ShapeDtypeStruct((1, 1), jnp.float32),
        ],
    )(x3, mus, musbf, sc3, lp3)

    riskflat = (gap[:, 0, :] < _DELTA).reshape(-1)
    rows = jnp.nonzero(riskflat, size=_S, fill_value=0)[0]
    bm_s = rows // hw
    t_s = rows % hw
    b_s = bm_s // _M
    m_s = bm_s % _M

    candF = cand.transpose(0, 2, 1).reshape(-1, 4)
    cands = candF[rows]
    cm = cands.T.reshape(-1)
    m_s4 = jnp.tile(m_s, 4)
    flat4 = m_s4 * _K + cm

    xt = x3.transpose(0, 2, 1).reshape(-1, c)
    xrow = xt[b_s * hw + t_s]

    mus2 = mus.reshape(_M * _K, _D)
    musg = mus2[flat4]
    musbfg = musbf.reshape(_M * _K, _D)[flat4]
    scck = jnp.clip(scales, _EPS, None)
    aux = jnp.concatenate([
        (2.0 * scck ** 2).reshape(-1, 1),
        jnp.log(scck).reshape(-1, 1),
        log_py_raw.reshape(-1, 1),
        jnp.zeros((_M * _K, 1), jnp.float32),
    ], axis=1)
    auxg = aux[flat4]

    win_rows = pl.pallas_call(
        _stage2_body,
        out_shape=jax.ShapeDtypeStruct((_S, _D), jnp.float32),
    )(xrow, m_s.reshape(-1, 1), musg, musbfg, auxg, cm.reshape(-1, 1))

    dcols_s = m_s[:, None] * _D + jnp.arange(_D)[None, :]
    sample3 = sample3.at[b_s[:, None], dcols_s, t_s[:, None]].set(win_rows)

    sample = sample3.reshape(b, c, h, w)
    kldesum = kld_acc[0, 0] / b
    return (sample, kldesum, jnp.zeros_like(kldesum))
